# baseline (speedup 1.0000x reference)
import functools

import jax
import jax.numpy as jnp
from jax import lax
from jax.experimental import pallas as pl
from jax.experimental.pallas import tpu as pltpu
from jax.experimental.pallas import tpu_sc as plsc


def _pe_add(x_flat, idx4, lpe, *, rows_per_w, chunk):
    n_chunks = rows_per_w // chunk
    R, D = x_flat.shape
    mesh = plsc.VectorSubcoreMesh(core_axis_name="c", subcore_axis_name="s")

    @functools.partial(
        pl.kernel,
        out_type=jax.ShapeDtypeStruct((R, D), jnp.float32),
        mesh=mesh,
        scratch_types=[
            pltpu.VMEM((chunk,), jnp.int32),
            pltpu.VMEM((chunk, D), jnp.float32),
            pltpu.VMEM((chunk, D), jnp.float32),
            pltpu.SemaphoreType.DMA,
        ],
    )
    def k(x_hbm, idx_hbm, lpe_hbm, out_hbm, idx_v, buf, pe_v, sem):
        wid = lax.axis_index("s") * 2 + lax.axis_index("c")
        base = wid * rows_per_w

        def body(c, carry):
            r0 = base + c * chunk
            pltpu.sync_copy(idx_hbm.at[pl.ds(r0, chunk)], idx_v)
            gather = pltpu.async_copy(lpe_hbm.at[idx_v], pe_v, sem)
            pltpu.sync_copy(x_hbm.at[pl.ds(r0, chunk)], buf)
            gather.wait()

            def add_row(r, carry2):
                for g in range(D // 16):
                    s = pl.ds(g * 16, 16)
                    buf[r, s] = buf[r, s] + pe_v[r, s]
                return carry2

            lax.fori_loop(0, chunk, add_row, 0)
            pltpu.sync_copy(buf, out_hbm.at[pl.ds(r0, chunk)])
            return carry

        lax.fori_loop(0, n_chunks, body, 0)

    return k(x_flat, idx4, lpe)


def kernel(x, indices, lpe):
    S, B, D = x.shape
    R = S * B
    x_flat = x.reshape(R, D)
    idx4 = jnp.repeat(indices.reshape(S).astype(jnp.int32), B)
    out = _pe_add(x_flat, idx4, lpe, rows_per_w=R // 32, chunk=32)
    return out.reshape(S, B, D)


# speedup vs baseline: 1.2500x; 1.2500x over previous
import functools

import jax
import jax.numpy as jnp
from jax import lax
from jax.experimental import pallas as pl
from jax.experimental.pallas import tpu as pltpu
from jax.experimental.pallas import tpu_sc as plsc

_NBUF = 3
_CH = 16


def _pe_add(x_flat, idx4, lpe, *, rows_per_w, chunk):
    n_chunks = rows_per_w // chunk
    R, D = x_flat.shape
    mesh = plsc.VectorSubcoreMesh(core_axis_name="c", subcore_axis_name="s")

    @functools.partial(
        pl.kernel,
        out_type=jax.ShapeDtypeStruct((R, D), jnp.float32),
        mesh=mesh,
        scratch_types=[
            pltpu.VMEM((rows_per_w,), jnp.int32),
            pltpu.VMEM((_NBUF, chunk, D), jnp.float32),
            pltpu.VMEM((_NBUF, chunk, D), jnp.float32),
            pltpu.SemaphoreType.DMA((_NBUF,)),
            pltpu.SemaphoreType.DMA((_NBUF,)),
        ],
    )
    def k(x_hbm, idx_hbm, lpe_hbm, out_hbm, idx_all, xbuf, pebuf, sem_in, sem_out):
        wid = lax.axis_index("s") * 2 + lax.axis_index("c")
        base = wid * rows_per_w
        pltpu.sync_copy(idx_hbm.at[pl.ds(base, rows_per_w)], idx_all)

        def issue_in(c, b):
            r0 = base + c * chunk
            pltpu.async_copy(x_hbm.at[pl.ds(r0, chunk)], xbuf.at[b], sem_in.at[b])
            pltpu.async_copy(
                lpe_hbm.at[idx_all.at[pl.ds(c * chunk, chunk)]],
                pebuf.at[b],
                sem_in.at[b],
            )

        def wait_in(b):
            pltpu.make_async_copy(x_hbm.at[pl.ds(0, chunk)], xbuf.at[b], sem_in.at[b]).wait()
            pltpu.make_async_copy(x_hbm.at[pl.ds(0, chunk)], pebuf.at[b], sem_in.at[b]).wait()

        def wait_out(b):
            pltpu.make_async_copy(xbuf.at[b], out_hbm.at[pl.ds(0, chunk)], sem_out.at[b]).wait()

        issue_in(0, 0)

        def step(c, carry):
            b = lax.rem(c, _NBUF)
            nxt = c + 1

            @pl.when(nxt < n_chunks)
            def _():
                @pl.when(c >= _NBUF - 1)
                def _():
                    wait_out(lax.rem(nxt, _NBUF))

                issue_in(nxt, lax.rem(nxt, _NBUF))

            wait_in(b)

            def add_row(r, carry2):
                for g in range(D // 16):
                    s = pl.ds(g * 16, 16)
                    xbuf[b, r, s] = xbuf[b, r, s] + pebuf[b, r, s]
                return carry2

            lax.fori_loop(0, chunk, add_row, 0)
            r0 = base + c * chunk
            pltpu.async_copy(xbuf.at[b], out_hbm.at[pl.ds(r0, chunk)], sem_out.at[b])
            return carry

        lax.fori_loop(0, n_chunks, step, 0)
        for k_last in range(n_chunks - _NBUF, n_chunks):
            wait_out(k_last % _NBUF)

    return k(x_flat, idx4, lpe)


def kernel(x, indices, lpe):
    S, B, D = x.shape
    R = S * B
    x_flat = x.reshape(R, D)
    idx4 = jnp.repeat(indices.reshape(S).astype(jnp.int32), B)
    out = _pe_add(x_flat, idx4, lpe, rows_per_w=R // 32, chunk=_CH)
    return out.reshape(S, B, D)


# speedup vs baseline: 3.8091x; 3.0473x over previous
import functools

import jax
import jax.numpy as jnp
from jax import lax
from jax.experimental import pallas as pl
from jax.experimental.pallas import tpu as pltpu
from jax.experimental.pallas import tpu_sc as plsc


def _sc_gather(idx, lpe, *, rows_per_w, chunk):
    n_chunks = rows_per_w // chunk
    S = idx.shape[0]
    D = lpe.shape[1]
    mesh = plsc.VectorSubcoreMesh(core_axis_name="c", subcore_axis_name="s")

    @functools.partial(
        pl.kernel,
        out_type=jax.ShapeDtypeStruct((S, D), jnp.float32),
        mesh=mesh,
        scratch_types=[
            pltpu.VMEM((rows_per_w,), jnp.int32),
            pltpu.VMEM((2, chunk, D), jnp.float32),
            pltpu.SemaphoreType.DMA((2,)),
            pltpu.SemaphoreType.DMA((2,)),
        ],
    )
    def k(idx_hbm, lpe_hbm, pe_hbm, idx_all, buf, sem_g, sem_o):
        wid = lax.axis_index("s") * 2 + lax.axis_index("c")
        base = wid * rows_per_w
        pltpu.sync_copy(idx_hbm.at[pl.ds(base, rows_per_w)], idx_all)

        def gather(c, b):
            pltpu.async_copy(
                lpe_hbm.at[idx_all.at[pl.ds(c * chunk, chunk)]],
                buf.at[b],
                sem_g.at[b],
            )

        def wait_gather(b):
            pltpu.make_async_copy(lpe_hbm.at[pl.ds(0, chunk)], buf.at[b], sem_g.at[b]).wait()

        def wait_out(b):
            pltpu.make_async_copy(buf.at[b], pe_hbm.at[pl.ds(0, chunk)], sem_o.at[b]).wait()

        gather(0, 0)

        def step(c, carry):
            b = lax.rem(c, 2)
            nxt = c + 1

            @pl.when(nxt < n_chunks)
            def _():
                @pl.when(c >= 1)
                def _():
                    wait_out(lax.rem(nxt, 2))

                gather(nxt, lax.rem(nxt, 2))

            wait_gather(b)
            pltpu.async_copy(buf.at[b], pe_hbm.at[pl.ds(base + c * chunk, chunk)], sem_o.at[b])
            return carry

        lax.fori_loop(0, n_chunks, step, 0)
        for c_last in range(max(0, n_chunks - 2), n_chunks):
            wait_out(c_last % 2)

    return k(idx, lpe)


def _tc_add(x, pe, *, bs):
    S, B, D = x.shape

    def body(x_ref, pe_ref, o_ref):
        o_ref[...] = x_ref[...] + pe_ref[...][:, None, :]

    return pl.pallas_call(
        body,
        grid=(S // bs,),
        in_specs=[
            pl.BlockSpec((bs, B, D), lambda i: (i, 0, 0)),
            pl.BlockSpec((bs, D), lambda i: (i, 0)),
        ],
        out_specs=pl.BlockSpec((bs, B, D), lambda i: (i, 0, 0)),
        out_shape=jax.ShapeDtypeStruct((S, B, D), jnp.float32),
    )(x, pe)


def kernel(x, indices, lpe):
    S, B, D = x.shape
    idx = indices.reshape(S).astype(jnp.int32)
    pe = _sc_gather(idx, lpe, rows_per_w=S // 32, chunk=32)
    return _tc_add(x, pe, bs=256)
